# baseline (speedup 1.0000x reference)
import functools

import jax
import jax.numpy as jnp
from jax import lax
from jax.experimental import pallas as pl
from jax.experimental.pallas import tpu as pltpu
from jax.experimental.pallas import tpu_sc as plsc

B = 16384
D = 64
L = 16
NC = 2
NS = 16
NW = NC * NS
BPW = B // NW
CH = 128
HALF = 256
NCH = HALF // CH
NG = HALF // L

NE = 1000000
NB = (NE + 127) // 128
NROW2 = NB * 64

_params = pltpu.CompilerParams(
    needs_layout_passes=False,
    use_tc_tiling_on_sc=True,
    disable_bounds_checks=True,
)
_mesh = plsc.VectorSubcoreMesh(core_axis_name="c", subcore_axis_name="s")


NTR = (NB + 1) // 2
NTRO = (NTR // NW + 1 + 3) // 4


@functools.partial(
    pl.kernel,
    mesh=_mesh,
    compiler_params=_params,
    out_type=(
        jax.ShapeDtypeStruct((NROW2, 128), jnp.float32),
        jax.ShapeDtypeStruct((NROW2, 128), jnp.float32),
    ),
    scratch_types=[
        pltpu.VMEM((4, D, 256), jnp.float32),
        pltpu.VMEM((2, 128, 128), jnp.float32),
        pltpu.SemaphoreType.DMA,
        pltpu.SemaphoreType.DMA,
        pltpu.SemaphoreType.DMA,
        pltpu.SemaphoreType.DMA,
        pltpu.SemaphoreType.DMA,
        pltpu.SemaphoreType.DMA,
    ],
)
def _repack_sc(entt_hbm, relt_hbm, e2_hbm, r2_hbm, tin_v, tout_v,
               si0, si1, si2, si3, so0, so1):
    wid = lax.axis_index("s") * NC + lax.axis_index("c")
    sis = (si0, si1, si2, si3)
    sos = (so0, so1)
    ntr = NTR // NW + jnp.where(wid < NTR % NW, 1, 0)
    tlo = (NTR // NW) * wid + jnp.minimum(wid, NTR % NW)

    lane = lax.iota(jnp.int32, L)

    for src_hbm, dst_hbm in ((entt_hbm, e2_hbm), (relt_hbm, r2_hbm)):

        def fire_in(tt, u, src_hbm=src_hbm):
            sb = jnp.minimum(2 * (tlo + tt), NB - 2)
            pltpu.async_copy(
                src_hbm.at[:, pl.ds(sb * 128, 256)], tin_v.at[u], sis[u]
            )

        for u in range(4):
            @pl.when(u < ntr)
            def _(u=u):
                fire_in(u, u)

        def o_body(o, carry, src_hbm=src_hbm, dst_hbm=dst_hbm):
            for u in range(4):
                tt = 4 * o + u
                os = u & 1

                @pl.when(tt < ntr)
                def _(tt=tt, u=u, os=os):
                    pltpu.make_async_copy(
                        src_hbm.at[:, pl.ds(0, 256)], tin_v.at[u], sis[u]
                    ).wait()

                    @pl.when(tt >= 2)
                    def _(os=os):
                        pltpu.make_async_copy(
                            dst_hbm.at[pl.ds(0, 128)], tout_v.at[os],
                            sos[os],
                        ).wait()

                    sb = jnp.minimum(2 * (tlo + tt), NB - 2)

                    def r_body(kp, _, u=u, os=os):
                        e0 = jnp.full((L,), 2 * kp, jnp.int32)
                        e1 = e0 + 1
                        uv = jnp.full((L,), u, jnp.int32)
                        for j in range(D // L):
                            dvec = j * L + lane
                            tout_v[os, kp, pl.ds(j * L, L)] = (
                                plsc.load_gather(tin_v, [uv, dvec, e0])
                            )
                            tout_v[os, kp, pl.ds(D + j * L, L)] = (
                                plsc.load_gather(tin_v, [uv, dvec, e1])
                            )
                        return 0

                    lax.fori_loop(0, 128, r_body, 0)

                    pltpu.async_copy(
                        tout_v.at[os], dst_hbm.at[pl.ds(sb * 64, 128)],
                        sos[os],
                    )

                    @pl.when(tt + 4 < ntr)
                    def _(tt=tt, u=u):
                        fire_in(tt + 4, u)

            return carry

        lax.fori_loop(0, NTRO, o_body, 0)

        for os in range(2):
            pltpu.make_async_copy(
                dst_hbm.at[pl.ds(0, 128)], tout_v.at[os], sos[os]
            ).wait()


@functools.partial(
    pl.kernel,
    mesh=_mesh,
    compiler_params=_params,
    out_type=jax.ShapeDtypeStruct((B,), jnp.float32),
    scratch_types=[
        pltpu.VMEM((BPW,), jnp.int32),
        pltpu.VMEM((BPW,), jnp.int32),
        pltpu.VMEM((BPW,), jnp.int32),
        pltpu.VMEM((NCH, CH), jnp.int32),
        pltpu.VMEM((NCH, CH), jnp.int32),
        pltpu.VMEM((NCH, CH), jnp.int32),
        pltpu.VMEM((HALF, 128), jnp.float32),
        pltpu.VMEM((HALF, 128), jnp.float32),
        pltpu.VMEM((HALF, 128), jnp.float32),
        pltpu.VMEM((BPW,), jnp.float32),
        pltpu.SemaphoreType.DMA,
    ],
)
def _transe_sc(hidx_hbm, ridx_hbm, tidx_hbm, ent2_hbm, rel2_hbm, out_hbm,
               hs_v, rs_v, ts_v, hk_v, rk_v, tk_v, hD_v, rD_v, tD_v,
               out_v, sem):
    wid = lax.axis_index("s") * NC + lax.axis_index("c")
    base = wid * BPW

    pltpu.sync_copy(hidx_hbm.at[pl.ds(base, BPW)], hs_v)
    pltpu.sync_copy(ridx_hbm.at[pl.ds(base, BPW)], rs_v)
    pltpu.sync_copy(tidx_hbm.at[pl.ds(base, BPW)], ts_v)

    lane = lax.iota(jnp.int32, L)

    for p in range(BPW // HALF):
        def i_body(g, carry):
            col = g * L
            off = p * HALF + col
            for st_v, k_v in ((hs_v, hk_v), (rs_v, rk_v), (ts_v, tk_v)):
                k_v[g >> 3, pl.ds((g & 7) * L, L)] = st_v[pl.ds(off, L)] >> 1
            return carry

        lax.fori_loop(0, NG, i_body, 0)

        cps = []
        for c in range(NCH):
            dst = pl.ds(c * CH, CH)
            cps.append(pltpu.async_copy(ent2_hbm.at[hk_v.at[c]], hD_v.at[dst], sem))
            cps.append(pltpu.async_copy(rel2_hbm.at[rk_v.at[c]], rD_v.at[dst], sem))
            cps.append(pltpu.async_copy(ent2_hbm.at[tk_v.at[c]], tD_v.at[dst], sem))
        for cp in cps:
            cp.wait()

        def c_body(g, carry):
            col = g * L
            off = p * HALF + col
            slots = col + lane
            hc = (hs_v[pl.ds(off, L)] & 1) * D
            rc = (rs_v[pl.ds(off, L)] & 1) * D
            tc = (ts_v[pl.ds(off, L)] & 1) * D

            def d_body(d, acc):
                hv = plsc.load_gather(hD_v, [slots, hc + d])
                rv = plsc.load_gather(rD_v, [slots, rc + d])
                tv = plsc.load_gather(tD_v, [slots, tc + d])
                return acc + jnp.abs(hv + rv - tv)

            acc = lax.fori_loop(0, D, d_body, jnp.zeros((L,), jnp.float32))
            out_v[pl.ds(off, L)] = acc
            return carry

        lax.fori_loop(0, NG, c_body, 0)

    pltpu.sync_copy(out_v, out_hbm.at[pl.ds(base, BPW)])


def kernel(triples, ent, rel):
    tr = triples.astype(jnp.int32)
    e2, r2 = _repack_sc(ent.T, rel.T)
    return _transe_sc(tr[:, 0], tr[:, 1], tr[:, 2], e2, r2)


# speedup vs baseline: 2.7318x; 2.7318x over previous
import functools

import jax
import jax.numpy as jnp
from jax import lax
from jax.experimental import pallas as pl
from jax.experimental.pallas import tpu as pltpu
from jax.experimental.pallas import tpu_sc as plsc

B = 16384
D = 64
L = 16
NC = 2
NS = 16
NW = NC * NS
BPW = B // NW
CH = 128
HALF = 256
NCH = HALF // CH
NG = HALF // L

NE = 1000000
NB = (NE + 127) // 128
NROW2 = NB * 64

_params = pltpu.CompilerParams(
    needs_layout_passes=False,
    use_tc_tiling_on_sc=True,
    disable_bounds_checks=True,
)
_mesh = plsc.VectorSubcoreMesh(core_axis_name="c", subcore_axis_name="s")


NTR = (NB + 1) // 2
NTRO = (NTR // NW + 1 + 3) // 4


@functools.partial(
    pl.kernel,
    mesh=_mesh,
    compiler_params=_params,
    out_type=(
        jax.ShapeDtypeStruct((NROW2, 128), jnp.float32),
        jax.ShapeDtypeStruct((NROW2, 128), jnp.float32),
    ),
    scratch_types=[
        pltpu.VMEM((4, D, 256), jnp.float32),
        pltpu.VMEM((2, 128, 128), jnp.float32),
        pltpu.SemaphoreType.DMA,
        pltpu.SemaphoreType.DMA,
        pltpu.SemaphoreType.DMA,
        pltpu.SemaphoreType.DMA,
        pltpu.SemaphoreType.DMA,
        pltpu.SemaphoreType.DMA,
    ],
)
def _repack_sc(entt_hbm, relt_hbm, e2_hbm, r2_hbm, tin_v, tout_v,
               si0, si1, si2, si3, so0, so1):
    wid = lax.axis_index("s") * NC + lax.axis_index("c")
    sis = (si0, si1, si2, si3)
    sos = (so0, so1)
    ntr = NTR // NW + jnp.where(wid < NTR % NW, 1, 0)
    tlo = (NTR // NW) * wid + jnp.minimum(wid, NTR % NW)

    lane = lax.iota(jnp.int32, L)

    for src_hbm, dst_hbm in ((entt_hbm, e2_hbm), (relt_hbm, r2_hbm)):

        def fire_in(tt, u, src_hbm=src_hbm):
            sb = jnp.minimum(2 * (tlo + tt), NB - 2)
            pltpu.async_copy(
                src_hbm.at[:, pl.ds(sb * 128, 256)], tin_v.at[u], sis[u]
            )

        for u in range(4):
            @pl.when(u < ntr)
            def _(u=u):
                fire_in(u, u)

        def o_body(o, carry, src_hbm=src_hbm, dst_hbm=dst_hbm):
            for u in range(4):
                tt = 4 * o + u
                os = u & 1

                @pl.when(tt < ntr)
                def _(tt=tt, u=u, os=os):
                    pltpu.make_async_copy(
                        src_hbm.at[:, pl.ds(0, 256)], tin_v.at[u], sis[u]
                    ).wait()

                    @pl.when(tt >= 2)
                    def _(os=os):
                        pltpu.make_async_copy(
                            dst_hbm.at[pl.ds(0, 128)], tout_v.at[os],
                            sos[os],
                        ).wait()

                    sb = jnp.minimum(2 * (tlo + tt), NB - 2)

                    def r_body(kp, _, u=u, os=os):
                        uv = jnp.full((L,), u, jnp.int32)
                        osv = jnp.full((L,), os, jnp.int32)
                        for j in range(D // L):
                            dvec = j * L + lane
                            for s in (0, 1):
                                evec = (2 * kp + s + lane) & 255
                                vals = plsc.load_gather(
                                    tin_v, [uv, dvec, evec]
                                )
                                rows = evec >> 1
                                cols = ((evec & 1) << 6) + dvec
                                plsc.store_scatter(
                                    tout_v, [osv, rows, cols], vals
                                )
                        return 0

                    lax.fori_loop(0, 128, r_body, 0)

                    pltpu.async_copy(
                        tout_v.at[os], dst_hbm.at[pl.ds(sb * 64, 128)],
                        sos[os],
                    )

                    @pl.when(tt + 4 < ntr)
                    def _(tt=tt, u=u):
                        fire_in(tt + 4, u)

            return carry

        lax.fori_loop(0, NTRO, o_body, 0)

        for os in range(2):
            pltpu.make_async_copy(
                dst_hbm.at[pl.ds(0, 128)], tout_v.at[os], sos[os]
            ).wait()


@functools.partial(
    pl.kernel,
    mesh=_mesh,
    compiler_params=_params,
    out_type=jax.ShapeDtypeStruct((B,), jnp.float32),
    scratch_types=[
        pltpu.VMEM((BPW,), jnp.int32),
        pltpu.VMEM((BPW,), jnp.int32),
        pltpu.VMEM((BPW,), jnp.int32),
        pltpu.VMEM((NCH, CH), jnp.int32),
        pltpu.VMEM((NCH, CH), jnp.int32),
        pltpu.VMEM((NCH, CH), jnp.int32),
        pltpu.VMEM((HALF, 128), jnp.float32),
        pltpu.VMEM((HALF, 128), jnp.float32),
        pltpu.VMEM((HALF, 128), jnp.float32),
        pltpu.VMEM((BPW,), jnp.float32),
        pltpu.SemaphoreType.DMA,
    ],
)
def _transe_sc(hidx_hbm, ridx_hbm, tidx_hbm, ent2_hbm, rel2_hbm, out_hbm,
               hs_v, rs_v, ts_v, hk_v, rk_v, tk_v, hD_v, rD_v, tD_v,
               out_v, sem):
    wid = lax.axis_index("s") * NC + lax.axis_index("c")
    base = wid * BPW

    pltpu.sync_copy(hidx_hbm.at[pl.ds(base, BPW)], hs_v)
    pltpu.sync_copy(ridx_hbm.at[pl.ds(base, BPW)], rs_v)
    pltpu.sync_copy(tidx_hbm.at[pl.ds(base, BPW)], ts_v)

    lane = lax.iota(jnp.int32, L)

    for p in range(BPW // HALF):
        def i_body(g, carry):
            col = g * L
            off = p * HALF + col
            for st_v, k_v in ((hs_v, hk_v), (rs_v, rk_v), (ts_v, tk_v)):
                k_v[g >> 3, pl.ds((g & 7) * L, L)] = st_v[pl.ds(off, L)] >> 1
            return carry

        lax.fori_loop(0, NG, i_body, 0)

        cps = []
        for c in range(NCH):
            dst = pl.ds(c * CH, CH)
            cps.append(pltpu.async_copy(ent2_hbm.at[hk_v.at[c]], hD_v.at[dst], sem))
            cps.append(pltpu.async_copy(rel2_hbm.at[rk_v.at[c]], rD_v.at[dst], sem))
            cps.append(pltpu.async_copy(ent2_hbm.at[tk_v.at[c]], tD_v.at[dst], sem))
        for cp in cps:
            cp.wait()

        def c_body(g, carry):
            col = g * L
            off = p * HALF + col
            slots = col + lane
            hc = (hs_v[pl.ds(off, L)] & 1) * D
            rc = (rs_v[pl.ds(off, L)] & 1) * D
            tc = (ts_v[pl.ds(off, L)] & 1) * D

            def d_body(d, acc):
                hv = plsc.load_gather(hD_v, [slots, hc + d])
                rv = plsc.load_gather(rD_v, [slots, rc + d])
                tv = plsc.load_gather(tD_v, [slots, tc + d])
                return acc + jnp.abs(hv + rv - tv)

            acc = lax.fori_loop(0, D, d_body, jnp.zeros((L,), jnp.float32))
            out_v[pl.ds(off, L)] = acc
            return carry

        lax.fori_loop(0, NG, c_body, 0)

    pltpu.sync_copy(out_v, out_hbm.at[pl.ds(base, BPW)])


def kernel(triples, ent, rel):
    tr = triples.astype(jnp.int32)
    e2, r2 = _repack_sc(ent.T, rel.T)
    return _transe_sc(tr[:, 0], tr[:, 1], tr[:, 2], e2, r2)


# speedup vs baseline: 2.8069x; 1.0275x over previous
import functools

import jax
import jax.numpy as jnp
from jax import lax
from jax.experimental import pallas as pl
from jax.experimental.pallas import tpu as pltpu
from jax.experimental.pallas import tpu_sc as plsc

B = 16384
D = 64
L = 16
NC = 2
NS = 16
NW = NC * NS
BPW = B // NW
CH = 128
HALF = 256
NCH = HALF // CH
NG = HALF // L

NE = 1000000
NB = (NE + 127) // 128
NROW2 = NB * 64

_params = pltpu.CompilerParams(
    needs_layout_passes=False,
    use_tc_tiling_on_sc=True,
    disable_bounds_checks=True,
)
_mesh = plsc.VectorSubcoreMesh(core_axis_name="c", subcore_axis_name="s")


NTR = (NB + 1) // 2
NTRO = (NTR // NW + 1 + 3) // 4


@functools.partial(
    pl.kernel,
    mesh=_mesh,
    compiler_params=_params,
    out_type=(
        jax.ShapeDtypeStruct((NROW2, 128), jnp.float32),
        jax.ShapeDtypeStruct((NROW2, 128), jnp.float32),
    ),
    scratch_types=[
        pltpu.VMEM((4, D, 256), jnp.float32),
        pltpu.VMEM((2, 128, 128), jnp.float32),
        pltpu.SemaphoreType.DMA,
        pltpu.SemaphoreType.DMA,
        pltpu.SemaphoreType.DMA,
        pltpu.SemaphoreType.DMA,
        pltpu.SemaphoreType.DMA,
        pltpu.SemaphoreType.DMA,
    ],
)
def _repack_sc(entt_hbm, relt_hbm, e2_hbm, r2_hbm, tin_v, tout_v,
               si0, si1, si2, si3, so0, so1):
    wid = lax.axis_index("s") * NC + lax.axis_index("c")
    sis = (si0, si1, si2, si3)
    sos = (so0, so1)
    ntr = NTR // NW + jnp.where(wid < NTR % NW, 1, 0)
    tlo = (NTR // NW) * wid + jnp.minimum(wid, NTR % NW)

    lane = lax.iota(jnp.int32, L)

    for src_hbm, dst_hbm in ((entt_hbm, e2_hbm), (relt_hbm, r2_hbm)):

        def fire_in(tt, u, src_hbm=src_hbm):
            sb = jnp.minimum(2 * (tlo + tt), NB - 2)
            pltpu.async_copy(
                src_hbm.at[:, pl.ds(sb * 128, 256)], tin_v.at[u], sis[u]
            )

        for u in range(4):
            @pl.when(u < ntr)
            def _(u=u):
                fire_in(u, u)

        def o_body(o, carry, src_hbm=src_hbm, dst_hbm=dst_hbm):
            for u in range(4):
                tt = 4 * o + u
                os = u & 1

                @pl.when(tt < ntr)
                def _(tt=tt, u=u, os=os):
                    pltpu.make_async_copy(
                        src_hbm.at[:, pl.ds(0, 256)], tin_v.at[u], sis[u]
                    ).wait()

                    @pl.when(tt >= 2)
                    def _(os=os):
                        pltpu.make_async_copy(
                            dst_hbm.at[pl.ds(0, 128)], tout_v.at[os],
                            sos[os],
                        ).wait()

                    sb = jnp.minimum(2 * (tlo + tt), NB - 2)

                    def r_body(kp, _, u=u, os=os):
                        uv = jnp.full((L,), u, jnp.int32)
                        osv = jnp.full((L,), os, jnp.int32)
                        for j in range(D // L):
                            dvec = j * L + lane
                            for s in (0, 1):
                                evec = (2 * kp + s + lane) & 255
                                vals = plsc.load_gather(
                                    tin_v, [uv, dvec, evec]
                                )
                                rows = evec >> 1
                                cols = ((evec & 1) << 6) + dvec
                                plsc.store_scatter(
                                    tout_v, [osv, rows, cols], vals
                                )
                        return 0

                    lax.fori_loop(0, 128, r_body, 0, unroll=4)

                    pltpu.async_copy(
                        tout_v.at[os], dst_hbm.at[pl.ds(sb * 64, 128)],
                        sos[os],
                    )

                    @pl.when(tt + 4 < ntr)
                    def _(tt=tt, u=u):
                        fire_in(tt + 4, u)

            return carry

        lax.fori_loop(0, NTRO, o_body, 0)

        for os in range(2):
            pltpu.make_async_copy(
                dst_hbm.at[pl.ds(0, 128)], tout_v.at[os], sos[os]
            ).wait()


@functools.partial(
    pl.kernel,
    mesh=_mesh,
    compiler_params=_params,
    out_type=jax.ShapeDtypeStruct((B,), jnp.float32),
    scratch_types=[
        pltpu.VMEM((BPW,), jnp.int32),
        pltpu.VMEM((BPW,), jnp.int32),
        pltpu.VMEM((BPW,), jnp.int32),
        pltpu.VMEM((NCH, CH), jnp.int32),
        pltpu.VMEM((NCH, CH), jnp.int32),
        pltpu.VMEM((NCH, CH), jnp.int32),
        pltpu.VMEM((HALF, 128), jnp.float32),
        pltpu.VMEM((HALF, 128), jnp.float32),
        pltpu.VMEM((HALF, 128), jnp.float32),
        pltpu.VMEM((BPW,), jnp.float32),
        pltpu.SemaphoreType.DMA,
    ],
)
def _transe_sc(hidx_hbm, ridx_hbm, tidx_hbm, ent2_hbm, rel2_hbm, out_hbm,
               hs_v, rs_v, ts_v, hk_v, rk_v, tk_v, hD_v, rD_v, tD_v,
               out_v, sem):
    wid = lax.axis_index("s") * NC + lax.axis_index("c")
    base = wid * BPW

    pltpu.sync_copy(hidx_hbm.at[pl.ds(base, BPW)], hs_v)
    pltpu.sync_copy(ridx_hbm.at[pl.ds(base, BPW)], rs_v)
    pltpu.sync_copy(tidx_hbm.at[pl.ds(base, BPW)], ts_v)

    lane = lax.iota(jnp.int32, L)

    for p in range(BPW // HALF):
        def i_body(g, carry):
            col = g * L
            off = p * HALF + col
            for st_v, k_v in ((hs_v, hk_v), (rs_v, rk_v), (ts_v, tk_v)):
                k_v[g >> 3, pl.ds((g & 7) * L, L)] = st_v[pl.ds(off, L)] >> 1
            return carry

        lax.fori_loop(0, NG, i_body, 0)

        cps = []
        for c in range(NCH):
            dst = pl.ds(c * CH, CH)
            cps.append(pltpu.async_copy(ent2_hbm.at[hk_v.at[c]], hD_v.at[dst], sem))
            cps.append(pltpu.async_copy(rel2_hbm.at[rk_v.at[c]], rD_v.at[dst], sem))
            cps.append(pltpu.async_copy(ent2_hbm.at[tk_v.at[c]], tD_v.at[dst], sem))
        for cp in cps:
            cp.wait()

        def c_body(g, carry):
            col = g * L
            off = p * HALF + col
            slots = col + lane
            hc = (hs_v[pl.ds(off, L)] & 1) * D
            rc = (rs_v[pl.ds(off, L)] & 1) * D
            tc = (ts_v[pl.ds(off, L)] & 1) * D

            def d_body(d, acc):
                hv = plsc.load_gather(hD_v, [slots, hc + d])
                rv = plsc.load_gather(rD_v, [slots, rc + d])
                tv = plsc.load_gather(tD_v, [slots, tc + d])
                return acc + jnp.abs(hv + rv - tv)

            acc = lax.fori_loop(0, D, d_body, jnp.zeros((L,), jnp.float32))
            out_v[pl.ds(off, L)] = acc
            return carry

        lax.fori_loop(0, NG, c_body, 0)

    pltpu.sync_copy(out_v, out_hbm.at[pl.ds(base, BPW)])


def kernel(triples, ent, rel):
    tr = triples.astype(jnp.int32)
    e2, r2 = _repack_sc(ent.T, rel.T)
    return _transe_sc(tr[:, 0], tr[:, 1], tr[:, 2], e2, r2)


# speedup vs baseline: 2.8218x; 1.0053x over previous
import functools

import jax
import jax.numpy as jnp
from jax import lax
from jax.experimental import pallas as pl
from jax.experimental.pallas import tpu as pltpu
from jax.experimental.pallas import tpu_sc as plsc

B = 16384
D = 64
L = 16
NC = 2
NS = 16
NW = NC * NS
BPW = B // NW
CH = 128
HALF = 256
NCH = HALF // CH
NG = HALF // L

NE = 1000000
NB = (NE + 127) // 128
NROW2 = NB * 64

_params = pltpu.CompilerParams(
    needs_layout_passes=False,
    use_tc_tiling_on_sc=True,
    disable_bounds_checks=True,
)
_mesh = plsc.VectorSubcoreMesh(core_axis_name="c", subcore_axis_name="s")


NTR = (NB + 1) // 2
NTRO = (NTR // NW + 1 + 3) // 4


@functools.partial(
    pl.kernel,
    mesh=_mesh,
    compiler_params=_params,
    out_type=(
        jax.ShapeDtypeStruct((NROW2, 128), jnp.float32),
        jax.ShapeDtypeStruct((NROW2, 128), jnp.float32),
    ),
    scratch_types=[
        pltpu.VMEM((4, D, 256), jnp.float32),
        pltpu.VMEM((2, 128, 128), jnp.float32),
        pltpu.SemaphoreType.DMA,
        pltpu.SemaphoreType.DMA,
        pltpu.SemaphoreType.DMA,
        pltpu.SemaphoreType.DMA,
        pltpu.SemaphoreType.DMA,
        pltpu.SemaphoreType.DMA,
    ],
)
def _repack_sc(entt_hbm, relt_hbm, e2_hbm, r2_hbm, tin_v, tout_v,
               si0, si1, si2, si3, so0, so1):
    wid = lax.axis_index("s") * NC + lax.axis_index("c")
    sis = (si0, si1, si2, si3)
    sos = (so0, so1)
    ntr = NTR // NW + jnp.where(wid < NTR % NW, 1, 0)
    tlo = (NTR // NW) * wid + jnp.minimum(wid, NTR % NW)

    lane = lax.iota(jnp.int32, L)

    for src_hbm, dst_hbm in ((entt_hbm, e2_hbm), (relt_hbm, r2_hbm)):

        def fire_in(tt, u, src_hbm=src_hbm):
            sb = jnp.minimum(2 * (tlo + tt), NB - 2)
            pltpu.async_copy(
                src_hbm.at[:, pl.ds(sb * 128, 256)], tin_v.at[u], sis[u]
            )

        for u in range(4):
            @pl.when(u < ntr)
            def _(u=u):
                fire_in(u, u)

        def o_body(o, carry, src_hbm=src_hbm, dst_hbm=dst_hbm):
            for u in range(4):
                tt = 4 * o + u
                os = u & 1

                @pl.when(tt < ntr)
                def _(tt=tt, u=u, os=os):
                    pltpu.make_async_copy(
                        src_hbm.at[:, pl.ds(0, 256)], tin_v.at[u], sis[u]
                    ).wait()

                    @pl.when(tt >= 2)
                    def _(os=os):
                        pltpu.make_async_copy(
                            dst_hbm.at[pl.ds(0, 128)], tout_v.at[os],
                            sos[os],
                        ).wait()

                    sb = jnp.minimum(2 * (tlo + tt), NB - 2)

                    def r_body(kp, _, u=u, os=os):
                        uv = jnp.full((L,), u, jnp.int32)
                        osv = jnp.full((L,), os, jnp.int32)
                        e0 = (2 * kp + lane) & 255
                        e1 = (e0 + 1) & 255
                        r0 = e0 >> 1
                        r1 = e1 >> 1
                        p0 = (e0 & 1) << 6
                        p1 = (e1 & 1) << 6
                        for j in range(D // L):
                            dvec = j * L + lane
                            v0 = plsc.load_gather(tin_v, [uv, dvec, e0])
                            plsc.store_scatter(
                                tout_v, [osv, r0, p0 + dvec], v0
                            )
                            v1 = plsc.load_gather(tin_v, [uv, dvec, e1])
                            plsc.store_scatter(
                                tout_v, [osv, r1, p1 + dvec], v1
                            )
                        return 0

                    lax.fori_loop(0, 128, r_body, 0, unroll=4)

                    pltpu.async_copy(
                        tout_v.at[os], dst_hbm.at[pl.ds(sb * 64, 128)],
                        sos[os],
                    )

                    @pl.when(tt + 4 < ntr)
                    def _(tt=tt, u=u):
                        fire_in(tt + 4, u)

            return carry

        lax.fori_loop(0, NTRO, o_body, 0)

        for os in range(2):
            pltpu.make_async_copy(
                dst_hbm.at[pl.ds(0, 128)], tout_v.at[os], sos[os]
            ).wait()


@functools.partial(
    pl.kernel,
    mesh=_mesh,
    compiler_params=_params,
    out_type=jax.ShapeDtypeStruct((B,), jnp.float32),
    scratch_types=[
        pltpu.VMEM((BPW,), jnp.int32),
        pltpu.VMEM((BPW,), jnp.int32),
        pltpu.VMEM((BPW,), jnp.int32),
        pltpu.VMEM((NCH, CH), jnp.int32),
        pltpu.VMEM((NCH, CH), jnp.int32),
        pltpu.VMEM((NCH, CH), jnp.int32),
        pltpu.VMEM((HALF, 128), jnp.float32),
        pltpu.VMEM((HALF, 128), jnp.float32),
        pltpu.VMEM((HALF, 128), jnp.float32),
        pltpu.VMEM((BPW,), jnp.float32),
        pltpu.SemaphoreType.DMA,
    ],
)
def _transe_sc(hidx_hbm, ridx_hbm, tidx_hbm, ent2_hbm, rel2_hbm, out_hbm,
               hs_v, rs_v, ts_v, hk_v, rk_v, tk_v, hD_v, rD_v, tD_v,
               out_v, sem):
    wid = lax.axis_index("s") * NC + lax.axis_index("c")
    base = wid * BPW

    pltpu.sync_copy(hidx_hbm.at[pl.ds(base, BPW)], hs_v)
    pltpu.sync_copy(ridx_hbm.at[pl.ds(base, BPW)], rs_v)
    pltpu.sync_copy(tidx_hbm.at[pl.ds(base, BPW)], ts_v)

    lane = lax.iota(jnp.int32, L)

    for p in range(BPW // HALF):
        def i_body(g, carry):
            col = g * L
            off = p * HALF + col
            for st_v, k_v in ((hs_v, hk_v), (rs_v, rk_v), (ts_v, tk_v)):
                k_v[g >> 3, pl.ds((g & 7) * L, L)] = st_v[pl.ds(off, L)] >> 1
            return carry

        lax.fori_loop(0, NG, i_body, 0)

        cps = []
        for c in range(NCH):
            dst = pl.ds(c * CH, CH)
            cps.append(pltpu.async_copy(ent2_hbm.at[hk_v.at[c]], hD_v.at[dst], sem))
            cps.append(pltpu.async_copy(rel2_hbm.at[rk_v.at[c]], rD_v.at[dst], sem))
            cps.append(pltpu.async_copy(ent2_hbm.at[tk_v.at[c]], tD_v.at[dst], sem))
        for cp in cps:
            cp.wait()

        def c_body(g, carry):
            col = g * L
            off = p * HALF + col
            slots = col + lane
            hc = (hs_v[pl.ds(off, L)] & 1) * D
            rc = (rs_v[pl.ds(off, L)] & 1) * D
            tc = (ts_v[pl.ds(off, L)] & 1) * D

            def d_body(d, acc):
                hv = plsc.load_gather(hD_v, [slots, hc + d])
                rv = plsc.load_gather(rD_v, [slots, rc + d])
                tv = plsc.load_gather(tD_v, [slots, tc + d])
                return acc + jnp.abs(hv + rv - tv)

            acc = lax.fori_loop(0, D, d_body, jnp.zeros((L,), jnp.float32))
            out_v[pl.ds(off, L)] = acc
            return carry

        lax.fori_loop(0, NG, c_body, 0)

    pltpu.sync_copy(out_v, out_hbm.at[pl.ds(base, BPW)])


def kernel(triples, ent, rel):
    tr = triples.astype(jnp.int32)
    e2, r2 = _repack_sc(ent.T, rel.T)
    return _transe_sc(tr[:, 0], tr[:, 1], tr[:, 2], e2, r2)
